# baseline (speedup 1.0000x reference)
import functools

import jax
import jax.numpy as jnp
import numpy as np
from jax import lax
from jax.experimental import pallas as pl
from jax.experimental.pallas import tpu as pltpu
from jax.experimental.pallas import tpu_sc as plsc

DIM_ = 32

_NP = 10240
_CH = 128
_NC = 2
_NS = 16
_NW = _NC * _NS


def _lrelu(x, s=0.01):
    return jnp.where(x > 0, x, s * x)


@functools.lru_cache(maxsize=None)
def _make_seg_sum(d, epad):
    et = epad // _NW
    nch = et // _CH
    assert nch % 2 == 0
    nzero = _NP // _CH // _NS
    mesh = plsc.VectorSubcoreMesh(core_axis_name="c", subcore_axis_name="s")

    @functools.partial(
        pl.kernel,
        out_type=jax.ShapeDtypeStruct((_NC, _NP, d), jnp.float32),
        mesh=mesh,
        scratch_types=[
            pltpu.VMEM((_CH,), jnp.int32),
            pltpu.VMEM((_CH,), jnp.int32),
            pltpu.VMEM((_CH, d), jnp.float32),
            pltpu.VMEM_SHARED((_NP, d), jnp.float32),
            pltpu.SemaphoreType.DMA,
        ],
        compiler_params=pltpu.CompilerParams(use_tc_tiling_on_sc=False),
    )
    def k(x_hbm, src_hbm, dst_hbm, out_hbm,
          src_v, dst_v, rows_v, acc, sem):
        c = lax.axis_index("c")
        s = lax.axis_index("s")
        wid = s * _NC + c

        zv = jnp.zeros((16,), jnp.float32)

        def zrow(r, _):
            for kk in range(d // 16):
                rows_v[r, pl.ds(kk * 16, 16)] = zv
            return 0

        lax.fori_loop(0, _CH, zrow, 0)
        for kk in range(nzero):
            chunk = s + kk * _NS
            pltpu.sync_copy(rows_v, acc.at[pl.ds(chunk * _CH, _CH)])
        plsc.subcore_barrier()

        def body(ci, _):
            base = wid * et + ci * _CH
            pltpu.sync_copy(src_hbm.at[pl.ds(base, _CH)], src_v)
            pltpu.sync_copy(dst_hbm.at[pl.ds(base, _CH)], dst_v)
            pltpu.async_copy(x_hbm.at[src_v], rows_v, sem).wait()
            pltpu.sync_copy(rows_v, acc.at[dst_v], add=True)
            return 0

        lax.fori_loop(0, nch, body, 0)
        plsc.subcore_barrier()
        for kk in range(nzero):
            chunk = s + kk * _NS
            pltpu.sync_copy(acc.at[pl.ds(chunk * _CH, _CH)],
                            out_hbm.at[c, pl.ds(chunk * _CH, _CH)])

    return k


@functools.lru_cache(maxsize=None)
def _make_gat_edge(epad):
    d = 144
    npa = 10112
    nacc = npa // _CH
    et = epad // _NW
    nch = et // _CH
    nzero = -(-nacc // _NS)
    mesh = plsc.VectorSubcoreMesh(core_axis_name="c", subcore_axis_name="s")

    @functools.partial(
        pl.kernel,
        out_type=jax.ShapeDtypeStruct((_NC, npa, d), jnp.float32),
        mesh=mesh,
        scratch_types=[
            pltpu.VMEM((_CH,), jnp.int32),
            pltpu.VMEM((_CH,), jnp.int32),
            pltpu.VMEM((_CH, d), jnp.float32),
            pltpu.VMEM((_NP,), jnp.float32),
            pltpu.VMEM((_NP,), jnp.float32),
            pltpu.VMEM((16,), jnp.float32),
            pltpu.VMEM_SHARED((npa, d), jnp.float32),
            pltpu.SemaphoreType.DMA,
        ],
        compiler_params=pltpu.CompilerParams(use_tc_tiling_on_sc=False,
                                             needs_layout_passes=False),
    )
    def k(hext_hbm, src_hbm, dst_hbm, as_hbm, ad_hbm, m_hbm, out_hbm,
          src_v, dst_v, rows_v, as_v, ad_v, m_v, acc, sem):
        c = lax.axis_index("c")
        s = lax.axis_index("s")
        wid = s * _NC + c

        pltpu.sync_copy(as_hbm, as_v)
        pltpu.sync_copy(ad_hbm, ad_v)
        pltpu.sync_copy(m_hbm, m_v)

        zv = jnp.zeros((16,), jnp.float32)

        def zrow(r, _):
            for kk in range(d // 16):
                rows_v[r, pl.ds(kk * 16, 16)] = zv
            return 0

        lax.fori_loop(0, _CH, zrow, 0)
        for kk in range(nzero):
            chunk = s + kk * _NS

            @pl.when(chunk < nacc)
            def _():
                pltpu.sync_copy(rows_v, acc.at[pl.ds(chunk * _CH, _CH)])
        plsc.subcore_barrier()

        def body(ci, _):
            base = wid * et + ci * _CH
            pltpu.sync_copy(src_hbm.at[pl.ds(base, _CH)], src_v)
            pltpu.sync_copy(dst_hbm.at[pl.ds(base, _CH)], dst_v)
            pltpu.async_copy(hext_hbm.at[src_v], rows_v, sem).wait()

            def sgroup(g, _):
                sv = src_v[pl.ds(g * 16, 16)]
                dv = dst_v[pl.ds(g * 16, 16)]
                asg = plsc.load_gather(as_v, [sv])
                adg = plsc.load_gather(ad_v, [dv])
                e = asg + adg
                e = jnp.where(e > 0, e, 0.2 * e) - m_v[...]
                exg = jnp.exp(e)
                for l in range(16):
                    exj = exg[l]
                    j = g * 16 + l
                    for kk in range(d // 16):
                        rows_v[j, pl.ds(kk * 16, 16)] = (
                            rows_v[j, pl.ds(kk * 16, 16)] * exj)
                return 0

            lax.fori_loop(0, _CH // 16, sgroup, 0)
            pltpu.sync_copy(rows_v, acc.at[dst_v], add=True)
            return 0

        lax.fori_loop(0, nch, body, 0)
        plsc.subcore_barrier()
        for kk in range(nzero):
            chunk = s + kk * _NS

            @pl.when(chunk < nacc)
            def _():
                pltpu.sync_copy(acc.at[pl.ds(chunk * _CH, _CH)],
                                out_hbm.at[c, pl.ds(chunk * _CH, _CH)])

    return k


def _seg_sum_sc(xp, srcp, dstp):
    parts = _make_seg_sum(xp.shape[1], srcp.shape[0])(xp, srcp, dstp)
    return parts[0] + parts[1]


def _pad_nodes(h):
    return jnp.pad(h, ((0, _NP - h.shape[0]), (0, 0)))


def _bn(x, p, eps=1e-5):
    mu = x.mean(0)
    var = x.var(0)
    return (x - mu) / jnp.sqrt(var + eps) * p['g'] + p['b']


_N = 10000


def _row_mask(shape):
    return (lax.broadcasted_iota(jnp.int32, shape, 0) < _N).astype(jnp.float32)


def _bn_masked(t, g, bb):
    maskf = _row_mask(t.shape)
    mu = jnp.sum(t * maskf, 0, keepdims=True) / _N
    td = (t - mu) * maskf
    var = jnp.sum(td * td, 0, keepdims=True) / _N
    return ((t - mu) / jnp.sqrt(var + 1e-5) * g + bb) * maskf


def _mm(a, b):
    return jnp.dot(a, b, preferred_element_type=jnp.float32)


def _stage1_body(p_ref, xp_ref, wl_ref, wr_ref, b_ref, g_ref, bb_ref,
                 h_ref, c_ref):
    sums = p_ref[0] + p_ref[1]
    cntc = jnp.maximum(sums[:, 19:20], 1.0)
    t = _mm(sums / cntc, wl_ref[...]) + _mm(xp_ref[...], wr_ref[...]) + b_ref[...]
    h_ref[...] = _bn_masked(_lrelu(t), g_ref[...], bb_ref[...])
    c_ref[...] = cntc


def _stagen_body(p_ref, hp_ref, c_ref, wl_ref, wr_ref, b_ref, g_ref, bb_ref,
                 h_ref):
    sums = p_ref[0] + p_ref[1]
    t = (_mm(sums / c_ref[...], wl_ref[...])
         + _mm(hp_ref[...], wr_ref[...]) + b_ref[...])
    h_ref[...] = _bn_masked(_lrelu(t), g_ref[...], bb_ref[...])


def _gat_prep_body(h_ref, w_ref, as_ref, ad_ref,
                   hext_ref, asp_ref, adp_ref, mv_ref):
    hw = _mm(h_ref[...], w_ref[...])
    a_s = _mm(hw, as_ref[...])
    a_d = _mm(hw, ad_ref[...])
    maskf = _row_mask(a_s.shape)
    neg = -1e30 * (1.0 - maskf)
    M = jnp.max(a_s + neg) + jnp.max(a_d + neg)
    M = jnp.where(M > 0, M, 0.2 * M)
    mv_ref[...] = jnp.zeros((1, 16), jnp.float32) + M
    hext_ref[...] = jnp.concatenate(
        [hw, maskf, jnp.zeros((hw.shape[0], 15), jnp.float32)], axis=1)
    asp_ref[...] = a_s
    adp_ref[...] = a_d


def _tail_a_body(p_ref, b1_ref, w0, b0, w1, b1l, w2, b2l, wg, asg, adg,
                 hw2_ref, as2_ref, ad2_ref):
    sums = p_ref[0] + p_ref[1]
    gval = sums[:, :128] / (sums[:, 128:129] + 1e-16) + b1_ref[...]
    g10 = gval[:_N]
    rr = lax.broadcasted_iota(jnp.int32, (200, _N), 0)
    cc = lax.broadcasted_iota(jnp.int32, (200, _N), 1)
    p1 = jnp.where(cc // 50 == rr, 1.0 / 50.0, 0.0)
    t = _mm(p1, g10)
    t = _lrelu(_mm(t, w0[...]) + b0[...])
    t = _lrelu(_mm(t, w1[...]) + b1l[...])
    t = _lrelu(_mm(t, w2[...]) + b2l[...])
    hw2 = _mm(t, wg[...])
    hw2_ref[...] = hw2
    as2_ref[...] = _mm(hw2, asg[...])
    ad2_ref[...] = _mm(hw2, adg[...])


def _tail_b_body(hw2_ref, as2_ref, adT_ref, b2_ref, w0, b0, w1, b1l, w2, b2l,
                 o_ref):
    C = 20
    e = as2_ref[...] + adT_ref[...]
    e = jnp.where(e > 0, e, 0.2 * e)
    ii = lax.broadcasted_iota(jnp.int32, (200, 200), 0)
    jj = lax.broadcasted_iota(jnp.int32, (200, 200), 1)
    samebatch = (ii // C) == (jj // C)
    em = jnp.where(samebatch, e, -1e30)
    m = jnp.max(em, axis=0, keepdims=True)
    ex = jnp.where(samebatch, jnp.exp(e - m), 0.0)
    ex = ex * jnp.where(ii == jj, 1.0, 2.0)
    z = jnp.sum(ex, axis=0, keepdims=True)
    coef = ex / (z + 1e-16)
    out200 = lax.dot_general(coef, hw2_ref[...], (((0,), (0,)), ((), ())),
                             preferred_element_type=jnp.float32)
    out200 = out200 + b2_ref[...]
    bb = lax.broadcasted_iota(jnp.int32, (10, 200), 0)
    nn = lax.broadcasted_iota(jnp.int32, (10, 200), 1)
    p2 = jnp.where(nn // C == bb, 1.0 / C, 0.0)
    t = _mm(p2, out200)
    t = _lrelu(_mm(t, w0[...]) + b0[...])
    t = _lrelu(_mm(t, w1[...]) + b1l[...])
    t = _lrelu(_lrelu(_mm(t, w2[...]) + b2l[...]))
    o_ref[...] = t


def kernel(x, params, edge_index, batch_size, num_cross, num_pieces):
    x = x.astype(jnp.float32)
    N = x.shape[0]
    src, dst = edge_index[0], edge_index[1]
    E = src.shape[0]

    gran = _NW * _CH * 2
    ep1 = ((E + gran - 1) // gran) * gran
    et1, nch1 = ep1 // _NW, ep1 // _NW // _CH
    srcp = jnp.concatenate([src, jnp.full((ep1 - E,), N, jnp.int32)])
    dstp = jnp.concatenate([dst, jnp.full((ep1 - E,), N, jnp.int32)])


    x20 = jnp.concatenate([x, jnp.ones((N, 1), jnp.float32)], axis=1)
    xp = jnp.pad(x20, ((0, _NP - N), (0, 12)))

    parts1 = _make_seg_sum(32, ep1)(xp, srcp, dstp)
    wl1 = jnp.pad(params['conv1']['Wl'], ((0, 13), (0, 0)))
    wr1 = jnp.pad(params['conv1']['Wr'], ((0, 13), (0, 0)))
    h, cntc = pl.pallas_call(
        _stage1_body,
        out_shape=[jax.ShapeDtypeStruct((_NP, 32), jnp.float32),
                   jax.ShapeDtypeStruct((_NP, 1), jnp.float32)],
    )(parts1, xp, wl1, wr1, params['conv1']['b'].reshape(1, -1),
      params['bn1']['g'].reshape(1, -1), params['bn1']['b'].reshape(1, -1))

    def stage(h, conv, bn):
        parts = _make_seg_sum(h.shape[1], ep1)(h, srcp, dstp)
        return pl.pallas_call(
            _stagen_body,
            out_shape=jax.ShapeDtypeStruct((_NP, conv['Wl'].shape[1]),
                                           jnp.float32),
        )(parts, h, cntc, conv['Wl'], conv['Wr'], conv['b'].reshape(1, -1),
          bn['g'].reshape(1, -1), bn['b'].reshape(1, -1))

    h = stage(h, params['conv2'], params['bn2'])
    h = stage(h, params['conv21'], params['bn21'])
    h = stage(h, params['conv22'], params['bn22'])
    h = stage(h, params['conv3'], params['bn3'])

    gp = params['gat1']
    hext, asp, adp, mvec = pl.pallas_call(
        _gat_prep_body,
        out_shape=[jax.ShapeDtypeStruct((_NP, 144), jnp.float32),
                   jax.ShapeDtypeStruct((_NP, 1), jnp.float32),
                   jax.ShapeDtypeStruct((_NP, 1), jnp.float32),
                   jax.ShapeDtypeStruct((1, 16), jnp.float32)],
    )(h, gp['W'], gp['as'].reshape(-1, 1), gp['ad'].reshape(-1, 1))

    ee = E + N
    ep2 = ((ee + gran - 1) // gran) * gran
    et2, nch2 = ep2 // _NW, ep2 // _NW // _CH
    loops = jnp.arange(N, dtype=src.dtype)
    esrc = jnp.concatenate([src, loops, jnp.full((ep2 - ee,), N, jnp.int32)])
    edst = jnp.concatenate([dst, loops, jnp.full((ep2 - ee,), N, jnp.int32)])
    parts_g = _make_gat_edge(ep2)(hext, esrc, edst, asp.reshape(-1),
                                  adp.reshape(-1), mvec.reshape(-1))

    l11 = params['lin11']
    g2 = params['gat2']
    hw2, as2, ad2 = pl.pallas_call(
        _tail_a_body,
        out_shape=[jax.ShapeDtypeStruct((200, 256), jnp.float32),
                   jax.ShapeDtypeStruct((200, 1), jnp.float32),
                   jax.ShapeDtypeStruct((200, 1), jnp.float32)],
    )(parts_g, gp['b'].reshape(1, -1),
      l11[0]['W'], l11[0]['b'].reshape(1, -1),
      l11[1]['W'], l11[1]['b'].reshape(1, -1),
      l11[2]['W'], l11[2]['b'].reshape(1, -1),
      g2['W'], g2['as'].reshape(-1, 1), g2['ad'].reshape(-1, 1))

    l12 = params['lin12']
    out = pl.pallas_call(
        _tail_b_body,
        out_shape=jax.ShapeDtypeStruct((10, 512), jnp.float32),
    )(hw2, as2, ad2.reshape(1, 200), g2['b'].reshape(1, -1),
      l12[0]['W'], l12[0]['b'].reshape(1, -1),
      l12[1]['W'], l12[1]['b'].reshape(1, -1),
      l12[2]['W'], l12[2]['b'].reshape(1, -1))
    return out


# speedup vs baseline: 1.4150x; 1.4150x over previous
import functools

import jax
import jax.numpy as jnp
import numpy as np
from jax import lax
from jax.experimental import pallas as pl
from jax.experimental.pallas import tpu as pltpu
from jax.experimental.pallas import tpu_sc as plsc

DIM_ = 32

_NP = 10240
_CH = 128
_NC = 2
_NS = 16
_NW = _NC * _NS


def _lrelu(x, s=0.01):
    return jnp.where(x > 0, x, s * x)


@functools.lru_cache(maxsize=None)
def _make_seg_sum(d, epad):
    et = epad // _NW
    nch = et // _CH
    nzero = _NP // _CH // _NS
    mesh = plsc.VectorSubcoreMesh(core_axis_name="c", subcore_axis_name="s")

    @functools.partial(
        pl.kernel,
        out_type=jax.ShapeDtypeStruct((_NC, _NP, d), jnp.float32),
        mesh=mesh,
        scratch_types=[
            pltpu.VMEM((_CH,), jnp.int32),
            pltpu.VMEM((_CH,), jnp.int32),
            pltpu.VMEM((_CH, d), jnp.float32),
            pltpu.VMEM_SHARED((_NP, d), jnp.float32),
            pltpu.SemaphoreType.DMA,
        ],
        compiler_params=pltpu.CompilerParams(use_tc_tiling_on_sc=False),
    )
    def k(x_hbm, src_hbm, dst_hbm, out_hbm,
          src_v, dst_v, rows_v, acc, sem):
        c = lax.axis_index("c")
        s = lax.axis_index("s")
        wid = s * _NC + c

        zv = jnp.zeros((16,), jnp.float32)

        def zrow(r, _):
            for kk in range(d // 16):
                rows_v[r, pl.ds(kk * 16, 16)] = zv
            return 0

        lax.fori_loop(0, _CH, zrow, 0)
        for kk in range(nzero):
            chunk = s + kk * _NS
            pltpu.sync_copy(rows_v, acc.at[pl.ds(chunk * _CH, _CH)])
        plsc.subcore_barrier()

        def body(ci, _):
            base = wid * et + ci * _CH
            pltpu.sync_copy(src_hbm.at[pl.ds(base, _CH)], src_v)
            pltpu.sync_copy(dst_hbm.at[pl.ds(base, _CH)], dst_v)
            pltpu.async_copy(x_hbm.at[src_v], rows_v, sem).wait()
            pltpu.sync_copy(rows_v, acc.at[dst_v], add=True)
            return 0

        lax.fori_loop(0, nch, body, 0)
        plsc.subcore_barrier()
        for kk in range(nzero):
            chunk = s + kk * _NS
            pltpu.sync_copy(acc.at[pl.ds(chunk * _CH, _CH)],
                            out_hbm.at[c, pl.ds(chunk * _CH, _CH)])

    return k


@functools.lru_cache(maxsize=None)
def _make_gat_edge(epad):
    d = 144
    npa = 10112
    nacc = npa // _CH
    et = epad // _NW
    nch = et // _CH
    nzero = -(-nacc // _NS)
    mesh = plsc.VectorSubcoreMesh(core_axis_name="c", subcore_axis_name="s")

    @functools.partial(
        pl.kernel,
        out_type=jax.ShapeDtypeStruct((_NC, npa, d), jnp.float32),
        mesh=mesh,
        scratch_types=[
            pltpu.VMEM((_CH,), jnp.int32),
            pltpu.VMEM((_CH,), jnp.int32),
            pltpu.VMEM((_CH, d), jnp.float32),
            pltpu.VMEM((_NP,), jnp.float32),
            pltpu.VMEM((_NP,), jnp.float32),
            pltpu.VMEM((16,), jnp.float32),
            pltpu.VMEM_SHARED((npa, d), jnp.float32),
            pltpu.SemaphoreType.DMA,
        ],
        compiler_params=pltpu.CompilerParams(use_tc_tiling_on_sc=False,
                                             needs_layout_passes=False),
    )
    def k(hext_hbm, src_hbm, dst_hbm, as_hbm, ad_hbm, m_hbm, out_hbm,
          src_v, dst_v, rows_v, as_v, ad_v, m_v, acc, sem):
        c = lax.axis_index("c")
        s = lax.axis_index("s")
        wid = s * _NC + c

        pltpu.sync_copy(as_hbm, as_v)
        pltpu.sync_copy(ad_hbm, ad_v)
        pltpu.sync_copy(m_hbm, m_v)

        zv = jnp.zeros((16,), jnp.float32)

        def zrow(r, _):
            for kk in range(d // 16):
                rows_v[r, pl.ds(kk * 16, 16)] = zv
            return 0

        lax.fori_loop(0, _CH, zrow, 0)
        for kk in range(nzero):
            chunk = s + kk * _NS

            @pl.when(chunk < nacc)
            def _():
                pltpu.sync_copy(rows_v, acc.at[pl.ds(chunk * _CH, _CH)])
        plsc.subcore_barrier()

        def body(ci, _):
            base = wid * et + ci * _CH
            pltpu.sync_copy(src_hbm.at[pl.ds(base, _CH)], src_v)
            pltpu.sync_copy(dst_hbm.at[pl.ds(base, _CH)], dst_v)
            pltpu.async_copy(hext_hbm.at[src_v], rows_v, sem).wait()

            def sgroup(g, _):
                sv = src_v[pl.ds(g * 16, 16)]
                dv = dst_v[pl.ds(g * 16, 16)]
                asg = plsc.load_gather(as_v, [sv])
                adg = plsc.load_gather(ad_v, [dv])
                e = asg + adg
                e = jnp.where(e > 0, e, 0.2 * e) - m_v[...]
                exg = jnp.exp(e)
                for l in range(16):
                    exj = exg[l]
                    j = g * 16 + l
                    for kk in range(d // 16):
                        rows_v[j, pl.ds(kk * 16, 16)] = (
                            rows_v[j, pl.ds(kk * 16, 16)] * exj)
                return 0

            lax.fori_loop(0, _CH // 16, sgroup, 0)
            pltpu.sync_copy(rows_v, acc.at[dst_v], add=True)
            return 0

        lax.fori_loop(0, nch, body, 0)
        plsc.subcore_barrier()
        for kk in range(nzero):
            chunk = s + kk * _NS

            @pl.when(chunk < nacc)
            def _():
                pltpu.sync_copy(acc.at[pl.ds(chunk * _CH, _CH)],
                                out_hbm.at[c, pl.ds(chunk * _CH, _CH)])

    return k


def _seg_sum_sc(xp, srcp, dstp):
    parts = _make_seg_sum(xp.shape[1], srcp.shape[0])(xp, srcp, dstp)
    return parts[0] + parts[1]


def _pad_nodes(h):
    return jnp.pad(h, ((0, _NP - h.shape[0]), (0, 0)))


def _bn(x, p, eps=1e-5):
    mu = x.mean(0)
    var = x.var(0)
    return (x - mu) / jnp.sqrt(var + eps) * p['g'] + p['b']


_N = 10000


def _row_mask(shape):
    return (lax.broadcasted_iota(jnp.int32, shape, 0) < _N).astype(jnp.float32)


def _bn_masked(t, g, bb):
    maskf = _row_mask(t.shape)
    mu = jnp.sum(t * maskf, 0, keepdims=True) / _N
    td = (t - mu) * maskf
    var = jnp.sum(td * td, 0, keepdims=True) / _N
    return ((t - mu) / jnp.sqrt(var + 1e-5) * g + bb) * maskf


def _mm(a, b):
    return jnp.dot(a, b, preferred_element_type=jnp.float32)


def _stage1_body(p_ref, xp_ref, wl_ref, wr_ref, b_ref, g_ref, bb_ref,
                 h_ref, c_ref):
    sums = p_ref[0] + p_ref[1]
    cntc = jnp.maximum(sums[:, 19:20], 1.0)
    t = _mm(sums / cntc, wl_ref[...]) + _mm(xp_ref[...], wr_ref[...]) + b_ref[...]
    h_ref[...] = _bn_masked(_lrelu(t), g_ref[...], bb_ref[...])
    c_ref[...] = cntc


def _stagen_body(p_ref, hp_ref, c_ref, wl_ref, wr_ref, b_ref, g_ref, bb_ref,
                 h_ref):
    sums = p_ref[0] + p_ref[1]
    t = (_mm(sums / c_ref[...], wl_ref[...])
         + _mm(hp_ref[...], wr_ref[...]) + b_ref[...])
    h_ref[...] = _bn_masked(_lrelu(t), g_ref[...], bb_ref[...])


def _gat_prep_body(h_ref, w_ref, as_ref, ad_ref,
                   hext_ref, asp_ref, adp_ref, mv_ref):
    hw = _mm(h_ref[...], w_ref[...])
    a_s = _mm(hw, as_ref[...])
    a_d = _mm(hw, ad_ref[...])
    maskf = _row_mask(a_s.shape)
    neg = -1e30 * (1.0 - maskf)
    M = jnp.max(a_s + neg) + jnp.max(a_d + neg)
    M = jnp.where(M > 0, M, 0.2 * M)
    mv_ref[...] = jnp.zeros((1, 16), jnp.float32) + M
    hext_ref[...] = jnp.concatenate(
        [hw, maskf, jnp.zeros((hw.shape[0], 15), jnp.float32)], axis=1)
    asp_ref[...] = a_s
    adp_ref[...] = a_d


def _tail_a_body(p_ref, b1_ref, w0, b0, w1, b1l, w2, b2l, wg, asg, adg,
                 hw2_ref, as2_ref, ad2_ref):
    sums = p_ref[0] + p_ref[1]
    gval = sums[:, :128] / (sums[:, 128:129] + 1e-16) + b1_ref[...]
    g10 = gval[:_N]
    rr = lax.broadcasted_iota(jnp.int32, (200, _N), 0)
    cc = lax.broadcasted_iota(jnp.int32, (200, _N), 1)
    p1 = jnp.where(cc // 50 == rr, 1.0 / 50.0, 0.0)
    t = _mm(p1, g10)
    t = _lrelu(_mm(t, w0[...]) + b0[...])
    t = _lrelu(_mm(t, w1[...]) + b1l[...])
    t = _lrelu(_mm(t, w2[...]) + b2l[...])
    hw2 = _mm(t, wg[...])
    hw2_ref[...] = hw2
    as2_ref[...] = _mm(hw2, asg[...])
    ad2_ref[...] = _mm(hw2, adg[...])


def _tail_b_body(hw2_ref, as2_ref, adT_ref, b2_ref, w0, b0, w1, b1l, w2, b2l,
                 o_ref):
    C = 20
    e = as2_ref[...] + adT_ref[...]
    e = jnp.where(e > 0, e, 0.2 * e)
    ii = lax.broadcasted_iota(jnp.int32, (200, 200), 0)
    jj = lax.broadcasted_iota(jnp.int32, (200, 200), 1)
    samebatch = (ii // C) == (jj // C)
    em = jnp.where(samebatch, e, -1e30)
    m = jnp.max(em, axis=0, keepdims=True)
    ex = jnp.where(samebatch, jnp.exp(e - m), 0.0)
    ex = ex * jnp.where(ii == jj, 1.0, 2.0)
    z = jnp.sum(ex, axis=0, keepdims=True)
    coef = ex / (z + 1e-16)
    out200 = lax.dot_general(coef, hw2_ref[...], (((0,), (0,)), ((), ())),
                             preferred_element_type=jnp.float32)
    out200 = out200 + b2_ref[...]
    bb = lax.broadcasted_iota(jnp.int32, (10, 200), 0)
    nn = lax.broadcasted_iota(jnp.int32, (10, 200), 1)
    p2 = jnp.where(nn // C == bb, 1.0 / C, 0.0)
    t = _mm(p2, out200)
    t = _lrelu(_mm(t, w0[...]) + b0[...])
    t = _lrelu(_mm(t, w1[...]) + b1l[...])
    t = _lrelu(_lrelu(_mm(t, w2[...]) + b2l[...]))
    o_ref[...] = t


def kernel(x, params, edge_index, batch_size, num_cross, num_pieces):
    x = x.astype(jnp.float32)
    N = x.shape[0]
    src, dst = edge_index[0], edge_index[1]
    E = src.shape[0]

    gran = _NW * _CH
    ep1 = ((E + gran - 1) // gran) * gran
    srcp = jnp.concatenate([src, jnp.full((ep1 - E,), N, jnp.int32)])
    dstp = jnp.concatenate([dst, jnp.full((ep1 - E,), N, jnp.int32)])


    x20 = jnp.concatenate([x, jnp.ones((N, 1), jnp.float32)], axis=1)
    xp = jnp.pad(x20, ((0, _NP - N), (0, 12)))

    parts1 = _make_seg_sum(32, ep1)(xp, srcp, dstp)
    wl1 = jnp.pad(params['conv1']['Wl'], ((0, 13), (0, 0)))
    wr1 = jnp.pad(params['conv1']['Wr'], ((0, 13), (0, 0)))
    h, cntc = pl.pallas_call(
        _stage1_body,
        out_shape=[jax.ShapeDtypeStruct((_NP, 32), jnp.float32),
                   jax.ShapeDtypeStruct((_NP, 1), jnp.float32)],
    )(parts1, xp, wl1, wr1, params['conv1']['b'].reshape(1, -1),
      params['bn1']['g'].reshape(1, -1), params['bn1']['b'].reshape(1, -1))

    def stage(h, conv, bn):
        parts = _make_seg_sum(h.shape[1], ep1)(h, srcp, dstp)
        return pl.pallas_call(
            _stagen_body,
            out_shape=jax.ShapeDtypeStruct((_NP, conv['Wl'].shape[1]),
                                           jnp.float32),
        )(parts, h, cntc, conv['Wl'], conv['Wr'], conv['b'].reshape(1, -1),
          bn['g'].reshape(1, -1), bn['b'].reshape(1, -1))

    h = stage(h, params['conv2'], params['bn2'])
    h = stage(h, params['conv21'], params['bn21'])
    h = stage(h, params['conv22'], params['bn22'])
    h = stage(h, params['conv3'], params['bn3'])

    gp = params['gat1']
    hext, asp, adp, mvec = pl.pallas_call(
        _gat_prep_body,
        out_shape=[jax.ShapeDtypeStruct((_NP, 144), jnp.float32),
                   jax.ShapeDtypeStruct((_NP, 1), jnp.float32),
                   jax.ShapeDtypeStruct((_NP, 1), jnp.float32),
                   jax.ShapeDtypeStruct((1, 16), jnp.float32)],
    )(h, gp['W'], gp['as'].reshape(-1, 1), gp['ad'].reshape(-1, 1))

    ee = E + N
    ep2 = ((ee + gran - 1) // gran) * gran
    loops = jnp.arange(N, dtype=src.dtype)
    esrc = jnp.concatenate([src, loops, jnp.full((ep2 - ee,), N, jnp.int32)])
    edst = jnp.concatenate([dst, loops, jnp.full((ep2 - ee,), N, jnp.int32)])
    parts_g = _make_gat_edge(ep2)(hext, esrc, edst, asp.reshape(-1),
                                  adp.reshape(-1), mvec.reshape(-1))

    l11 = params['lin11']
    g2 = params['gat2']
    hw2, as2, ad2 = pl.pallas_call(
        _tail_a_body,
        out_shape=[jax.ShapeDtypeStruct((200, 256), jnp.float32),
                   jax.ShapeDtypeStruct((200, 1), jnp.float32),
                   jax.ShapeDtypeStruct((200, 1), jnp.float32)],
    )(parts_g, gp['b'].reshape(1, -1),
      l11[0]['W'], l11[0]['b'].reshape(1, -1),
      l11[1]['W'], l11[1]['b'].reshape(1, -1),
      l11[2]['W'], l11[2]['b'].reshape(1, -1),
      g2['W'], g2['as'].reshape(-1, 1), g2['ad'].reshape(-1, 1))

    l12 = params['lin12']
    out = pl.pallas_call(
        _tail_b_body,
        out_shape=jax.ShapeDtypeStruct((10, 512), jnp.float32),
    )(hw2, as2, ad2.reshape(1, 200), g2['b'].reshape(1, -1),
      l12[0]['W'], l12[0]['b'].reshape(1, -1),
      l12[1]['W'], l12[1]['b'].reshape(1, -1),
      l12[2]['W'], l12[2]['b'].reshape(1, -1))
    return out


# speedup vs baseline: 1.4639x; 1.0345x over previous
import functools

import jax
import jax.numpy as jnp
import numpy as np
from jax import lax
from jax.experimental import pallas as pl
from jax.experimental.pallas import tpu as pltpu
from jax.experimental.pallas import tpu_sc as plsc

DIM_ = 32

_NP = 10240
_CH = 128
_NC = 2
_NS = 16
_NW = _NC * _NS


def _lrelu(x, s=0.01):
    return jnp.where(x > 0, x, s * x)


@functools.lru_cache(maxsize=None)
def _make_seg_sum(d, epad):
    et = epad // _NW
    nch = et // _CH
    nzero = _NP // _CH // _NS
    mesh = plsc.VectorSubcoreMesh(core_axis_name="c", subcore_axis_name="s")

    @functools.partial(
        pl.kernel,
        out_type=jax.ShapeDtypeStruct((_NC, _NP, d), jnp.float32),
        mesh=mesh,
        scratch_types=[
            pltpu.VMEM((_CH,), jnp.int32),
            pltpu.VMEM((_CH,), jnp.int32),
            pltpu.VMEM((_CH, d), jnp.float32),
            pltpu.VMEM_SHARED((_NP, d), jnp.float32),
            pltpu.SemaphoreType.DMA,
        ],
        compiler_params=pltpu.CompilerParams(use_tc_tiling_on_sc=False),
    )
    def k(x_hbm, src_hbm, dst_hbm, out_hbm,
          src_v, dst_v, rows_v, acc, sem):
        c = lax.axis_index("c")
        s = lax.axis_index("s")
        wid = s * _NC + c

        zv = jnp.zeros((16,), jnp.float32)

        def zrow(r, _):
            for kk in range(d // 16):
                rows_v[r, pl.ds(kk * 16, 16)] = zv
            return 0

        lax.fori_loop(0, _CH, zrow, 0)
        for kk in range(nzero):
            chunk = s + kk * _NS
            pltpu.sync_copy(rows_v, acc.at[pl.ds(chunk * _CH, _CH)])
        plsc.subcore_barrier()

        def body(ci, _):
            base = wid * et + ci * _CH
            pltpu.sync_copy(src_hbm.at[pl.ds(base, _CH)], src_v)
            pltpu.sync_copy(dst_hbm.at[pl.ds(base, _CH)], dst_v)
            pltpu.async_copy(x_hbm.at[src_v], rows_v, sem).wait()
            pltpu.sync_copy(rows_v, acc.at[dst_v], add=True)
            return 0

        lax.fori_loop(0, nch, body, 0)
        plsc.subcore_barrier()
        for kk in range(nzero):
            chunk = s + kk * _NS
            pltpu.sync_copy(acc.at[pl.ds(chunk * _CH, _CH)],
                            out_hbm.at[c, pl.ds(chunk * _CH, _CH)])

    return k


@functools.lru_cache(maxsize=None)
def _make_gat_edge(epad):
    d = 144
    ch = 64
    npa = 10112
    nacc = npa // _CH
    et = epad // _NW
    nch = et // ch
    assert nch % 2 == 0
    nzero = -(-nacc // _NS)
    mesh = plsc.VectorSubcoreMesh(core_axis_name="c", subcore_axis_name="s")

    @functools.partial(
        pl.kernel,
        out_type=jax.ShapeDtypeStruct((_NC, npa, d), jnp.float32),
        mesh=mesh,
        scratch_types=[
            pltpu.VMEM((ch,), jnp.int32),
            pltpu.VMEM((ch,), jnp.int32),
            pltpu.VMEM((ch,), jnp.int32),
            pltpu.VMEM((ch,), jnp.int32),
            pltpu.VMEM((ch, d), jnp.float32),
            pltpu.VMEM((ch, d), jnp.float32),
            pltpu.VMEM((_NP,), jnp.float32),
            pltpu.VMEM((_NP,), jnp.float32),
            pltpu.VMEM((16,), jnp.float32),
            pltpu.VMEM_SHARED((npa, d), jnp.float32),
            pltpu.SemaphoreType.DMA,
            pltpu.SemaphoreType.DMA,
        ],
        compiler_params=pltpu.CompilerParams(use_tc_tiling_on_sc=False,
                                             needs_layout_passes=False),
    )
    def k(hext_hbm, src_hbm, dst_hbm, as_hbm, ad_hbm, m_hbm, out_hbm,
          srcv0, srcv1, dstv0, dstv1, rows0, rows1, as_v, ad_v, m_v, acc,
          semg0, semg1):
        c = lax.axis_index("c")
        s = lax.axis_index("s")
        wid = s * _NC + c

        pltpu.sync_copy(as_hbm, as_v)
        pltpu.sync_copy(ad_hbm, ad_v)
        pltpu.sync_copy(m_hbm, m_v)

        zv = jnp.zeros((16,), jnp.float32)

        def zrow(r, _):
            for kk in range(d // 16):
                rows0[r, pl.ds(kk * 16, 16)] = zv
            return 0

        lax.fori_loop(0, ch, zrow, 0)
        for kk in range(nzero * 2):
            chunk = s + kk * _NS

            @pl.when(chunk * ch < npa)
            def _():
                pltpu.sync_copy(rows0, acc.at[pl.ds(chunk * ch, ch)])
        plsc.subcore_barrier()

        def fetch(ci, srcv, dstv, rows, semg):
            base = wid * et + ci * ch
            pltpu.sync_copy(src_hbm.at[pl.ds(base, ch)], srcv)
            pltpu.sync_copy(dst_hbm.at[pl.ds(base, ch)], dstv)
            pltpu.async_copy(hext_hbm.at[srcv], rows, semg)

        def process(srcv, dstv, rows, semg):
            pltpu.make_async_copy(hext_hbm.at[srcv], rows, semg).wait()

            def sgroup(g, _):
                sv = srcv[pl.ds(g * 16, 16)]
                dv = dstv[pl.ds(g * 16, 16)]
                asg = plsc.load_gather(as_v, [sv])
                adg = plsc.load_gather(ad_v, [dv])
                e = asg + adg
                e = jnp.where(e > 0, e, 0.2 * e) - m_v[...]
                exg = jnp.exp(e)
                for l in range(16):
                    exj = exg[l]
                    j = g * 16 + l
                    for kk in range(d // 16):
                        rows[j, pl.ds(kk * 16, 16)] = (
                            rows[j, pl.ds(kk * 16, 16)] * exj)
                return 0

            lax.fori_loop(0, ch // 16, sgroup, 0)
            pltpu.sync_copy(rows, acc.at[dstv], add=True)

        fetch(0, srcv0, dstv0, rows0, semg0)

        def body(i2, _):
            i = 2 * i2
            fetch(i + 1, srcv1, dstv1, rows1, semg1)
            process(srcv0, dstv0, rows0, semg0)

            @pl.when(i + 2 < nch)
            def _():
                fetch(i + 2, srcv0, dstv0, rows0, semg0)

            process(srcv1, dstv1, rows1, semg1)
            return 0

        lax.fori_loop(0, nch // 2, body, 0)
        plsc.subcore_barrier()
        for kk in range(nzero):
            chunk = s + kk * _NS

            @pl.when(chunk < nacc)
            def _():
                pltpu.sync_copy(acc.at[pl.ds(chunk * _CH, _CH)],
                                out_hbm.at[c, pl.ds(chunk * _CH, _CH)])

    return k


def _seg_sum_sc(xp, srcp, dstp):
    parts = _make_seg_sum(xp.shape[1], srcp.shape[0])(xp, srcp, dstp)
    return parts[0] + parts[1]


def _pad_nodes(h):
    return jnp.pad(h, ((0, _NP - h.shape[0]), (0, 0)))


def _bn(x, p, eps=1e-5):
    mu = x.mean(0)
    var = x.var(0)
    return (x - mu) / jnp.sqrt(var + eps) * p['g'] + p['b']


_N = 10000


def _row_mask(shape):
    return (lax.broadcasted_iota(jnp.int32, shape, 0) < _N).astype(jnp.float32)


def _bn_masked(t, g, bb):
    maskf = _row_mask(t.shape)
    mu = jnp.sum(t * maskf, 0, keepdims=True) / _N
    td = (t - mu) * maskf
    var = jnp.sum(td * td, 0, keepdims=True) / _N
    return ((t - mu) / jnp.sqrt(var + 1e-5) * g + bb) * maskf


def _mm(a, b):
    return jnp.dot(a, b, preferred_element_type=jnp.float32)


def _stage1_body(p_ref, xp_ref, wl_ref, wr_ref, b_ref, g_ref, bb_ref,
                 h_ref, c_ref):
    sums = p_ref[0] + p_ref[1]
    cntc = jnp.maximum(sums[:, 19:20], 1.0)
    t = _mm(sums / cntc, wl_ref[...]) + _mm(xp_ref[...], wr_ref[...]) + b_ref[...]
    h_ref[...] = _bn_masked(_lrelu(t), g_ref[...], bb_ref[...])
    c_ref[...] = cntc


def _stagen_body(p_ref, hp_ref, c_ref, wl_ref, wr_ref, b_ref, g_ref, bb_ref,
                 h_ref):
    sums = p_ref[0] + p_ref[1]
    t = (_mm(sums / c_ref[...], wl_ref[...])
         + _mm(hp_ref[...], wr_ref[...]) + b_ref[...])
    h_ref[...] = _bn_masked(_lrelu(t), g_ref[...], bb_ref[...])


def _gat_prep_body(h_ref, w_ref, as_ref, ad_ref,
                   hext_ref, asp_ref, adp_ref, mv_ref):
    hw = _mm(h_ref[...], w_ref[...])
    a_s = _mm(hw, as_ref[...])
    a_d = _mm(hw, ad_ref[...])
    maskf = _row_mask(a_s.shape)
    neg = -1e30 * (1.0 - maskf)
    M = jnp.max(a_s + neg) + jnp.max(a_d + neg)
    M = jnp.where(M > 0, M, 0.2 * M)
    mv_ref[...] = jnp.zeros((1, 16), jnp.float32) + M
    hext_ref[...] = jnp.concatenate(
        [hw, maskf, jnp.zeros((hw.shape[0], 15), jnp.float32)], axis=1)
    asp_ref[...] = a_s
    adp_ref[...] = a_d


def _tail_a_body(p_ref, b1_ref, w0, b0, w1, b1l, w2, b2l, wg, asg, adg,
                 hw2_ref, as2_ref, ad2_ref):
    sums = p_ref[0] + p_ref[1]
    gval = sums[:, :128] / (sums[:, 128:129] + 1e-16) + b1_ref[...]
    g10 = gval[:_N]
    rr = lax.broadcasted_iota(jnp.int32, (200, _N), 0)
    cc = lax.broadcasted_iota(jnp.int32, (200, _N), 1)
    p1 = jnp.where(cc // 50 == rr, 1.0 / 50.0, 0.0)
    t = _mm(p1, g10)
    t = _lrelu(_mm(t, w0[...]) + b0[...])
    t = _lrelu(_mm(t, w1[...]) + b1l[...])
    t = _lrelu(_mm(t, w2[...]) + b2l[...])
    hw2 = _mm(t, wg[...])
    hw2_ref[...] = hw2
    as2_ref[...] = _mm(hw2, asg[...])
    ad2_ref[...] = _mm(hw2, adg[...])


def _tail_b_body(hw2_ref, as2_ref, adT_ref, b2_ref, w0, b0, w1, b1l, w2, b2l,
                 o_ref):
    C = 20
    e = as2_ref[...] + adT_ref[...]
    e = jnp.where(e > 0, e, 0.2 * e)
    ii = lax.broadcasted_iota(jnp.int32, (200, 200), 0)
    jj = lax.broadcasted_iota(jnp.int32, (200, 200), 1)
    samebatch = (ii // C) == (jj // C)
    em = jnp.where(samebatch, e, -1e30)
    m = jnp.max(em, axis=0, keepdims=True)
    ex = jnp.where(samebatch, jnp.exp(e - m), 0.0)
    ex = ex * jnp.where(ii == jj, 1.0, 2.0)
    z = jnp.sum(ex, axis=0, keepdims=True)
    coef = ex / (z + 1e-16)
    out200 = lax.dot_general(coef, hw2_ref[...], (((0,), (0,)), ((), ())),
                             preferred_element_type=jnp.float32)
    out200 = out200 + b2_ref[...]
    bb = lax.broadcasted_iota(jnp.int32, (10, 200), 0)
    nn = lax.broadcasted_iota(jnp.int32, (10, 200), 1)
    p2 = jnp.where(nn // C == bb, 1.0 / C, 0.0)
    t = _mm(p2, out200)
    t = _lrelu(_mm(t, w0[...]) + b0[...])
    t = _lrelu(_mm(t, w1[...]) + b1l[...])
    t = _lrelu(_lrelu(_mm(t, w2[...]) + b2l[...]))
    o_ref[...] = t


def kernel(x, params, edge_index, batch_size, num_cross, num_pieces):
    x = x.astype(jnp.float32)
    N = x.shape[0]
    src, dst = edge_index[0], edge_index[1]
    E = src.shape[0]

    gran = _NW * _CH
    ep1 = ((E + gran - 1) // gran) * gran
    srcp = jnp.concatenate([src, jnp.full((ep1 - E,), N, jnp.int32)])
    dstp = jnp.concatenate([dst, jnp.full((ep1 - E,), N, jnp.int32)])


    x20 = jnp.concatenate([x, jnp.ones((N, 1), jnp.float32)], axis=1)
    xp = jnp.pad(x20, ((0, _NP - N), (0, 12)))

    parts1 = _make_seg_sum(32, ep1)(xp, srcp, dstp)
    wl1 = jnp.pad(params['conv1']['Wl'], ((0, 13), (0, 0)))
    wr1 = jnp.pad(params['conv1']['Wr'], ((0, 13), (0, 0)))
    h, cntc = pl.pallas_call(
        _stage1_body,
        out_shape=[jax.ShapeDtypeStruct((_NP, 32), jnp.float32),
                   jax.ShapeDtypeStruct((_NP, 1), jnp.float32)],
    )(parts1, xp, wl1, wr1, params['conv1']['b'].reshape(1, -1),
      params['bn1']['g'].reshape(1, -1), params['bn1']['b'].reshape(1, -1))

    def stage(h, conv, bn):
        parts = _make_seg_sum(h.shape[1], ep1)(h, srcp, dstp)
        return pl.pallas_call(
            _stagen_body,
            out_shape=jax.ShapeDtypeStruct((_NP, conv['Wl'].shape[1]),
                                           jnp.float32),
        )(parts, h, cntc, conv['Wl'], conv['Wr'], conv['b'].reshape(1, -1),
          bn['g'].reshape(1, -1), bn['b'].reshape(1, -1))

    h = stage(h, params['conv2'], params['bn2'])
    h = stage(h, params['conv21'], params['bn21'])
    h = stage(h, params['conv22'], params['bn22'])
    h = stage(h, params['conv3'], params['bn3'])

    gp = params['gat1']
    hext, asp, adp, mvec = pl.pallas_call(
        _gat_prep_body,
        out_shape=[jax.ShapeDtypeStruct((_NP, 144), jnp.float32),
                   jax.ShapeDtypeStruct((_NP, 1), jnp.float32),
                   jax.ShapeDtypeStruct((_NP, 1), jnp.float32),
                   jax.ShapeDtypeStruct((1, 16), jnp.float32)],
    )(h, gp['W'], gp['as'].reshape(-1, 1), gp['ad'].reshape(-1, 1))

    ee = E + N
    ep2 = ((ee + gran - 1) // gran) * gran
    loops = jnp.arange(N, dtype=src.dtype)
    esrc = jnp.concatenate([src, loops, jnp.full((ep2 - ee,), N, jnp.int32)])
    edst = jnp.concatenate([dst, loops, jnp.full((ep2 - ee,), N, jnp.int32)])
    parts_g = _make_gat_edge(ep2)(hext, esrc, edst, asp.reshape(-1),
                                  adp.reshape(-1), mvec.reshape(-1))

    l11 = params['lin11']
    g2 = params['gat2']
    hw2, as2, ad2 = pl.pallas_call(
        _tail_a_body,
        out_shape=[jax.ShapeDtypeStruct((200, 256), jnp.float32),
                   jax.ShapeDtypeStruct((200, 1), jnp.float32),
                   jax.ShapeDtypeStruct((200, 1), jnp.float32)],
    )(parts_g, gp['b'].reshape(1, -1),
      l11[0]['W'], l11[0]['b'].reshape(1, -1),
      l11[1]['W'], l11[1]['b'].reshape(1, -1),
      l11[2]['W'], l11[2]['b'].reshape(1, -1),
      g2['W'], g2['as'].reshape(-1, 1), g2['ad'].reshape(-1, 1))

    l12 = params['lin12']
    out = pl.pallas_call(
        _tail_b_body,
        out_shape=jax.ShapeDtypeStruct((10, 512), jnp.float32),
    )(hw2, as2, ad2.reshape(1, 200), g2['b'].reshape(1, -1),
      l12[0]['W'], l12[0]['b'].reshape(1, -1),
      l12[1]['W'], l12[1]['b'].reshape(1, -1),
      l12[2]['W'], l12[2]['b'].reshape(1, -1))
    return out


# speedup vs baseline: 1.8921x; 1.2925x over previous
import functools

import jax
import jax.numpy as jnp
import numpy as np
from jax import lax
from jax.experimental import pallas as pl
from jax.experimental.pallas import tpu as pltpu
from jax.experimental.pallas import tpu_sc as plsc

DIM_ = 32

_NP = 10240
_CH = 128
_NC = 2
_NS = 16
_NW = _NC * _NS


def _lrelu(x, s=0.01):
    return jnp.where(x > 0, x, s * x)


@functools.lru_cache(maxsize=None)
def _make_seg_sum(d, epad):
    et = epad // _NW
    nch = et // _CH
    nzero = _NP // _CH // _NS
    mesh = plsc.VectorSubcoreMesh(core_axis_name="c", subcore_axis_name="s")

    @functools.partial(
        pl.kernel,
        out_type=jax.ShapeDtypeStruct((_NC, _NP, d), jnp.float32),
        mesh=mesh,
        scratch_types=[
            pltpu.VMEM((_CH,), jnp.int32),
            pltpu.VMEM((_CH,), jnp.int32),
            pltpu.VMEM((_CH,), jnp.int32),
            pltpu.VMEM((_CH,), jnp.int32),
            pltpu.VMEM((_CH, d), jnp.float32),
            pltpu.VMEM((_CH, d), jnp.float32),
            pltpu.VMEM_SHARED((_NP, d), jnp.float32),
            pltpu.SemaphoreType.DMA,
            pltpu.SemaphoreType.DMA,
        ],
        compiler_params=pltpu.CompilerParams(use_tc_tiling_on_sc=False),
    )
    def k(x_hbm, src_hbm, dst_hbm, out_hbm,
          srcv0, srcv1, dstv0, dstv1, rows0, rows1, acc, semg0, semg1):
        c = lax.axis_index("c")
        s = lax.axis_index("s")
        wid = s * _NC + c

        zv = jnp.zeros((16,), jnp.float32)

        def zrow(r, _):
            for kk in range(d // 16):
                rows0[r, pl.ds(kk * 16, 16)] = zv
            return 0

        lax.fori_loop(0, _CH, zrow, 0)
        for kk in range(nzero):
            chunk = s + kk * _NS
            pltpu.sync_copy(rows0, acc.at[pl.ds(chunk * _CH, _CH)])
        plsc.subcore_barrier()

        def fetch(ci, srcv, dstv, rows, semg):
            base = wid * et + ci * _CH
            pltpu.sync_copy(src_hbm.at[pl.ds(base, _CH)], srcv)
            pltpu.sync_copy(dst_hbm.at[pl.ds(base, _CH)], dstv)
            pltpu.async_copy(x_hbm.at[srcv], rows, semg)

        def process(srcv, dstv, rows, semg):
            pltpu.make_async_copy(x_hbm.at[srcv], rows, semg).wait()
            pltpu.sync_copy(rows, acc.at[dstv], add=True)

        assert nch % 2 == 1
        fetch(0, srcv0, dstv0, rows0, semg0)

        def body(i2, _):
            i = 2 * i2
            fetch(i + 1, srcv1, dstv1, rows1, semg1)
            process(srcv0, dstv0, rows0, semg0)
            fetch(i + 2, srcv0, dstv0, rows0, semg0)
            process(srcv1, dstv1, rows1, semg1)
            return 0

        lax.fori_loop(0, (nch - 1) // 2, body, 0)
        process(srcv0, dstv0, rows0, semg0)
        plsc.subcore_barrier()
        for kk in range(nzero):
            chunk = s + kk * _NS
            pltpu.sync_copy(acc.at[pl.ds(chunk * _CH, _CH)],
                            out_hbm.at[c, pl.ds(chunk * _CH, _CH)])

    return k


@functools.lru_cache(maxsize=None)
def _make_gat_edge(epad):
    d = 144
    ch = 64
    npa = 10112
    nacc = npa // _CH
    et = epad // _NW
    nch = et // ch
    assert nch % 2 == 0
    nzero = -(-nacc // _NS)
    mesh = plsc.VectorSubcoreMesh(core_axis_name="c", subcore_axis_name="s")

    @functools.partial(
        pl.kernel,
        out_type=jax.ShapeDtypeStruct((_NC, npa, d), jnp.float32),
        mesh=mesh,
        scratch_types=[
            pltpu.VMEM((ch,), jnp.int32),
            pltpu.VMEM((ch,), jnp.int32),
            pltpu.VMEM((ch,), jnp.int32),
            pltpu.VMEM((ch,), jnp.int32),
            pltpu.VMEM((ch, d), jnp.float32),
            pltpu.VMEM((ch, d), jnp.float32),
            pltpu.VMEM((_NP,), jnp.float32),
            pltpu.VMEM((_NP,), jnp.float32),
            pltpu.VMEM((16,), jnp.float32),
            pltpu.VMEM_SHARED((npa, d), jnp.float32),
            pltpu.SemaphoreType.DMA,
            pltpu.SemaphoreType.DMA,
        ],
        compiler_params=pltpu.CompilerParams(use_tc_tiling_on_sc=False,
                                             needs_layout_passes=False),
    )
    def k(hext_hbm, src_hbm, dst_hbm, as_hbm, ad_hbm, m_hbm, out_hbm,
          srcv0, srcv1, dstv0, dstv1, rows0, rows1, as_v, ad_v, m_v, acc,
          semg0, semg1):
        c = lax.axis_index("c")
        s = lax.axis_index("s")
        wid = s * _NC + c

        pltpu.sync_copy(as_hbm, as_v)
        pltpu.sync_copy(ad_hbm, ad_v)
        pltpu.sync_copy(m_hbm, m_v)

        zv = jnp.zeros((16,), jnp.float32)

        def zrow(r, _):
            for kk in range(d // 16):
                rows0[r, pl.ds(kk * 16, 16)] = zv
            return 0

        lax.fori_loop(0, ch, zrow, 0)
        for kk in range(nzero * 2):
            chunk = s + kk * _NS

            @pl.when(chunk * ch < npa)
            def _():
                pltpu.sync_copy(rows0, acc.at[pl.ds(chunk * ch, ch)])
        plsc.subcore_barrier()

        def fetch(ci, srcv, dstv, rows, semg):
            base = wid * et + ci * ch
            pltpu.sync_copy(src_hbm.at[pl.ds(base, ch)], srcv)
            pltpu.sync_copy(dst_hbm.at[pl.ds(base, ch)], dstv)
            pltpu.async_copy(hext_hbm.at[srcv], rows, semg)

        def process(srcv, dstv, rows, semg):
            pltpu.make_async_copy(hext_hbm.at[srcv], rows, semg).wait()

            def sgroup(g, _):
                sv = srcv[pl.ds(g * 16, 16)]
                dv = dstv[pl.ds(g * 16, 16)]
                asg = plsc.load_gather(as_v, [sv])
                adg = plsc.load_gather(ad_v, [dv])
                e = asg + adg
                e = jnp.where(e > 0, e, 0.2 * e) - m_v[...]
                exg = jnp.exp(e)
                for l in range(16):
                    exj = exg[l]
                    j = g * 16 + l
                    for kk in range(d // 16):
                        rows[j, pl.ds(kk * 16, 16)] = (
                            rows[j, pl.ds(kk * 16, 16)] * exj)
                return 0

            lax.fori_loop(0, ch // 16, sgroup, 0)
            pltpu.sync_copy(rows, acc.at[dstv], add=True)

        fetch(0, srcv0, dstv0, rows0, semg0)

        def body(i2, _):
            i = 2 * i2
            fetch(i + 1, srcv1, dstv1, rows1, semg1)
            process(srcv0, dstv0, rows0, semg0)

            @pl.when(i + 2 < nch)
            def _():
                fetch(i + 2, srcv0, dstv0, rows0, semg0)

            process(srcv1, dstv1, rows1, semg1)
            return 0

        lax.fori_loop(0, nch // 2, body, 0)
        plsc.subcore_barrier()
        for kk in range(nzero):
            chunk = s + kk * _NS

            @pl.when(chunk < nacc)
            def _():
                pltpu.sync_copy(acc.at[pl.ds(chunk * _CH, _CH)],
                                out_hbm.at[c, pl.ds(chunk * _CH, _CH)])

    return k


def _seg_sum_sc(xp, srcp, dstp):
    parts = _make_seg_sum(xp.shape[1], srcp.shape[0])(xp, srcp, dstp)
    return parts[0] + parts[1]


def _pad_nodes(h):
    return jnp.pad(h, ((0, _NP - h.shape[0]), (0, 0)))


def _bn(x, p, eps=1e-5):
    mu = x.mean(0)
    var = x.var(0)
    return (x - mu) / jnp.sqrt(var + eps) * p['g'] + p['b']


_N = 10000


def _row_mask(shape):
    return (lax.broadcasted_iota(jnp.int32, shape, 0) < _N).astype(jnp.float32)


def _bn_masked(t, g, bb):
    maskf = _row_mask(t.shape)
    mu = jnp.sum(t * maskf, 0, keepdims=True) / _N
    td = (t - mu) * maskf
    var = jnp.sum(td * td, 0, keepdims=True) / _N
    return ((t - mu) / jnp.sqrt(var + 1e-5) * g + bb) * maskf


def _mm(a, b):
    return jnp.dot(a, b, preferred_element_type=jnp.float32)


def _stage1_body(p_ref, xp_ref, wl_ref, wr_ref, b_ref, g_ref, bb_ref,
                 h_ref, c_ref):
    sums = p_ref[0] + p_ref[1]
    cntc = jnp.maximum(sums[:, 19:20], 1.0)
    t = _mm(sums / cntc, wl_ref[...]) + _mm(xp_ref[...], wr_ref[...]) + b_ref[...]
    h_ref[...] = _bn_masked(_lrelu(t), g_ref[...], bb_ref[...])
    c_ref[...] = cntc


def _stagen_body(p_ref, hp_ref, c_ref, wl_ref, wr_ref, b_ref, g_ref, bb_ref,
                 h_ref):
    sums = p_ref[0] + p_ref[1]
    t = (_mm(sums / c_ref[...], wl_ref[...])
         + _mm(hp_ref[...], wr_ref[...]) + b_ref[...])
    h_ref[...] = _bn_masked(_lrelu(t), g_ref[...], bb_ref[...])


def _gat_prep_body(h_ref, w_ref, as_ref, ad_ref,
                   hext_ref, asp_ref, adp_ref, mv_ref):
    hw = _mm(h_ref[...], w_ref[...])
    a_s = _mm(hw, as_ref[...])
    a_d = _mm(hw, ad_ref[...])
    maskf = _row_mask(a_s.shape)
    neg = -1e30 * (1.0 - maskf)
    M = jnp.max(a_s + neg) + jnp.max(a_d + neg)
    M = jnp.where(M > 0, M, 0.2 * M)
    mv_ref[...] = jnp.zeros((1, 16), jnp.float32) + M
    hext_ref[...] = jnp.concatenate(
        [hw, maskf, jnp.zeros((hw.shape[0], 15), jnp.float32)], axis=1)
    asp_ref[...] = a_s
    adp_ref[...] = a_d


def _tail_a_body(p_ref, b1_ref, w0, b0, w1, b1l, w2, b2l, wg, asg, adg,
                 hw2_ref, as2_ref, ad2_ref):
    sums = p_ref[0] + p_ref[1]
    gval = sums[:, :128] / (sums[:, 128:129] + 1e-16) + b1_ref[...]
    g10 = gval[:_N]
    rr = lax.broadcasted_iota(jnp.int32, (200, _N), 0)
    cc = lax.broadcasted_iota(jnp.int32, (200, _N), 1)
    p1 = jnp.where(cc // 50 == rr, 1.0 / 50.0, 0.0)
    t = _mm(p1, g10)
    t = _lrelu(_mm(t, w0[...]) + b0[...])
    t = _lrelu(_mm(t, w1[...]) + b1l[...])
    t = _lrelu(_mm(t, w2[...]) + b2l[...])
    hw2 = _mm(t, wg[...])
    hw2_ref[...] = hw2
    as2_ref[...] = _mm(hw2, asg[...])
    ad2_ref[...] = _mm(hw2, adg[...])


def _tail_b_body(hw2_ref, as2_ref, adT_ref, b2_ref, w0, b0, w1, b1l, w2, b2l,
                 o_ref):
    C = 20
    e = as2_ref[...] + adT_ref[...]
    e = jnp.where(e > 0, e, 0.2 * e)
    ii = lax.broadcasted_iota(jnp.int32, (200, 200), 0)
    jj = lax.broadcasted_iota(jnp.int32, (200, 200), 1)
    samebatch = (ii // C) == (jj // C)
    em = jnp.where(samebatch, e, -1e30)
    m = jnp.max(em, axis=0, keepdims=True)
    ex = jnp.where(samebatch, jnp.exp(e - m), 0.0)
    ex = ex * jnp.where(ii == jj, 1.0, 2.0)
    z = jnp.sum(ex, axis=0, keepdims=True)
    coef = ex / (z + 1e-16)
    out200 = lax.dot_general(coef, hw2_ref[...], (((0,), (0,)), ((), ())),
                             preferred_element_type=jnp.float32)
    out200 = out200 + b2_ref[...]
    bb = lax.broadcasted_iota(jnp.int32, (10, 200), 0)
    nn = lax.broadcasted_iota(jnp.int32, (10, 200), 1)
    p2 = jnp.where(nn // C == bb, 1.0 / C, 0.0)
    t = _mm(p2, out200)
    t = _lrelu(_mm(t, w0[...]) + b0[...])
    t = _lrelu(_mm(t, w1[...]) + b1l[...])
    t = _lrelu(_lrelu(_mm(t, w2[...]) + b2l[...]))
    o_ref[...] = t


def kernel(x, params, edge_index, batch_size, num_cross, num_pieces):
    x = x.astype(jnp.float32)
    N = x.shape[0]
    src, dst = edge_index[0], edge_index[1]
    E = src.shape[0]

    gran = _NW * _CH
    ep1 = ((E + gran - 1) // gran) * gran
    srcp = jnp.concatenate([src, jnp.full((ep1 - E,), N, jnp.int32)])
    dstp = jnp.concatenate([dst, jnp.full((ep1 - E,), N, jnp.int32)])


    x20 = jnp.concatenate([x, jnp.ones((N, 1), jnp.float32)], axis=1)
    xp = jnp.pad(x20, ((0, _NP - N), (0, 12)))

    parts1 = _make_seg_sum(32, ep1)(xp, srcp, dstp)
    wl1 = jnp.pad(params['conv1']['Wl'], ((0, 13), (0, 0)))
    wr1 = jnp.pad(params['conv1']['Wr'], ((0, 13), (0, 0)))
    h, cntc = pl.pallas_call(
        _stage1_body,
        out_shape=[jax.ShapeDtypeStruct((_NP, 32), jnp.float32),
                   jax.ShapeDtypeStruct((_NP, 1), jnp.float32)],
    )(parts1, xp, wl1, wr1, params['conv1']['b'].reshape(1, -1),
      params['bn1']['g'].reshape(1, -1), params['bn1']['b'].reshape(1, -1))

    def stage(h, conv, bn):
        parts = _make_seg_sum(h.shape[1], ep1)(h, srcp, dstp)
        return pl.pallas_call(
            _stagen_body,
            out_shape=jax.ShapeDtypeStruct((_NP, conv['Wl'].shape[1]),
                                           jnp.float32),
        )(parts, h, cntc, conv['Wl'], conv['Wr'], conv['b'].reshape(1, -1),
          bn['g'].reshape(1, -1), bn['b'].reshape(1, -1))

    h = stage(h, params['conv2'], params['bn2'])
    h = stage(h, params['conv21'], params['bn21'])
    h = stage(h, params['conv22'], params['bn22'])
    h = stage(h, params['conv3'], params['bn3'])

    gp = params['gat1']
    hext, asp, adp, mvec = pl.pallas_call(
        _gat_prep_body,
        out_shape=[jax.ShapeDtypeStruct((_NP, 144), jnp.float32),
                   jax.ShapeDtypeStruct((_NP, 1), jnp.float32),
                   jax.ShapeDtypeStruct((_NP, 1), jnp.float32),
                   jax.ShapeDtypeStruct((1, 16), jnp.float32)],
    )(h, gp['W'], gp['as'].reshape(-1, 1), gp['ad'].reshape(-1, 1))

    ee = E + N
    ep2 = ((ee + gran - 1) // gran) * gran
    loops = jnp.arange(N, dtype=src.dtype)
    esrc = jnp.concatenate([src, loops, jnp.full((ep2 - ee,), N, jnp.int32)])
    edst = jnp.concatenate([dst, loops, jnp.full((ep2 - ee,), N, jnp.int32)])
    parts_g = _make_gat_edge(ep2)(hext, esrc, edst, asp.reshape(-1),
                                  adp.reshape(-1), mvec.reshape(-1))

    l11 = params['lin11']
    g2 = params['gat2']
    hw2, as2, ad2 = pl.pallas_call(
        _tail_a_body,
        out_shape=[jax.ShapeDtypeStruct((200, 256), jnp.float32),
                   jax.ShapeDtypeStruct((200, 1), jnp.float32),
                   jax.ShapeDtypeStruct((200, 1), jnp.float32)],
    )(parts_g, gp['b'].reshape(1, -1),
      l11[0]['W'], l11[0]['b'].reshape(1, -1),
      l11[1]['W'], l11[1]['b'].reshape(1, -1),
      l11[2]['W'], l11[2]['b'].reshape(1, -1),
      g2['W'], g2['as'].reshape(-1, 1), g2['ad'].reshape(-1, 1))

    l12 = params['lin12']
    out = pl.pallas_call(
        _tail_b_body,
        out_shape=jax.ShapeDtypeStruct((10, 512), jnp.float32),
    )(hw2, as2, ad2.reshape(1, 200), g2['b'].reshape(1, -1),
      l12[0]['W'], l12[0]['b'].reshape(1, -1),
      l12[1]['W'], l12[1]['b'].reshape(1, -1),
      l12[2]['W'], l12[2]['b'].reshape(1, -1))
    return out
